# baseline (speedup 1.0000x reference)
import functools

import jax
import jax.numpy as jnp
from jax import lax
from jax.experimental import pallas as pl
from jax.experimental.pallas import tpu as pltpu
from jax.experimental.pallas import tpu_sc as plsc

NUM_TASKS = 3
PROMPT_LEN = 20
HIDDEN = 4096
BATCH = 1024

WIDE = 1024
R_EL = PROMPT_LEN * HIDDEN // WIDE
HALF = R_EL // 2
ROWS = BATCH * R_EL
T_ROWS = NUM_TASKS * R_EL

NUM_CORES = 2
NUM_SUBCORES = 16
NUM_WORKERS = NUM_CORES * NUM_SUBCORES

B_PER_TILE = BATCH // NUM_WORKERS
N_B = 12
FLIGHT = 12


def _sc_lookup(task_ids, table2):
    mesh = plsc.VectorSubcoreMesh(core_axis_name="c", subcore_axis_name="s")

    @functools.partial(
        pl.kernel,
        out_type=jax.ShapeDtypeStruct((ROWS, WIDE), jnp.float32),
        mesh=mesh,
        scratch_types=[
            pltpu.VMEM((B_PER_TILE,), jnp.int32),
            pltpu.VMEM((HALF, WIDE), jnp.float32),
            pltpu.VMEM((HALF, WIDE), jnp.float32),
            pltpu.VMEM_SHARED((T_ROWS, WIDE), jnp.float32),
            pltpu.SemaphoreType.DMA,
            pltpu.SemaphoreType.DMA,
            pltpu.SemaphoreType.DMA,
        ],
    )
    def run(ids_hbm, table2_hbm, out_hbm, idx_v, tb0, tb1, sh_table,
            asem, ssem0, ssem1):
        c = lax.axis_index("c")
        s = lax.axis_index("s")
        wid = s * NUM_CORES + c
        base_el = wid * B_PER_TILE
        base_row = base_el * R_EL

        pltpu.sync_copy(ids_hbm.at[pl.ds(base_el, B_PER_TILE)], idx_v)

        @pl.when(s == 0)
        def _():
            pltpu.sync_copy(table2_hbm, sh_table)

        plsc.subcore_barrier()

        vecs = [idx_v[pl.ds(0, 16)], idx_v[pl.ds(16, 16)]]

        def tid_of(e):
            return vecs[e // 16][e % 16]

        tbufs = (tb0, tb1)
        ssems = (ssem0, ssem1)

        def scat_wait(q):
            pltpu.make_async_copy(
                tbufs[q], out_hbm.at[pl.ds(base_row, HALF)], ssems[q]).wait()

        n_a = 0
        a_burst = (B_PER_TILE - N_B + N_B - 1) // N_B

        def issue_a(e):
            pltpu.async_copy(
                sh_table.at[pl.ds(tid_of(e) * R_EL, R_EL)],
                out_hbm.at[pl.ds(base_row + e * R_EL, R_EL)], asem)

        a_next = N_B
        for eb in range(N_B):
            tid = tid_of(eb)
            for h in range(2):
                j = 2 * eb + h
                q = j % 2
                if j >= 2:
                    scat_wait(q)
                pltpu.sync_copy(
                    sh_table.at[pl.ds(tid * R_EL + h * HALF, HALF)],
                    tbufs[q])
                pltpu.async_copy(
                    tbufs[q],
                    out_hbm.at[pl.ds(base_row + eb * R_EL + h * HALF, HALF)],
                    ssems[q])
            for _ in range(a_burst):
                if a_next < B_PER_TILE:
                    issue_a(a_next)
                    a_next += 1
                    n_a += 1
        while a_next < B_PER_TILE:
            issue_a(a_next)
            a_next += 1
            n_a += 1

        scat_wait(0)
        scat_wait(1)
        for _ in range(n_a):
            pltpu.make_async_copy(
                sh_table.at[pl.ds(0, R_EL)],
                out_hbm.at[pl.ds(base_row, R_EL)], asem).wait()

    return run(task_ids, table2)


def kernel(task_ids, prompt_embeddings):
    ids = task_ids.astype(jnp.int32)
    table2 = prompt_embeddings.reshape(T_ROWS, WIDE)
    out = _sc_lookup(ids, table2)
    return out.reshape(BATCH, PROMPT_LEN, HIDDEN)


# speedup vs baseline: 1.5966x; 1.5966x over previous
import functools

import jax
import jax.numpy as jnp
from jax import lax
from jax.experimental import pallas as pl
from jax.experimental.pallas import tpu as pltpu
from jax.experimental.pallas import tpu_sc as plsc

NUM_TASKS = 3
PROMPT_LEN = 20
HIDDEN = 4096
BATCH = 1024

B_TC = 768
B_SC = BATCH - B_TC

NUM_CORES = 2
NUM_SUBCORES = 16
NUM_WORKERS = NUM_CORES * NUM_SUBCORES

B_PER_TILE = B_SC // NUM_WORKERS
BLOCK_B = 8


def _tc_main(task_ids, table):
    def body(ids_ref, table_ref, out_ref):
        b0 = pl.program_id(0) * BLOCK_B
        for i in range(BLOCK_B):
            tid = ids_ref[b0 + i]
            out_ref[i] = table_ref[tid]

    grid_spec = pltpu.PrefetchScalarGridSpec(
        num_scalar_prefetch=1,
        grid=(B_TC // BLOCK_B,),
        in_specs=[
            pl.BlockSpec((NUM_TASKS, PROMPT_LEN, HIDDEN),
                         lambda b, ids: (0, 0, 0)),
        ],
        out_specs=pl.BlockSpec((BLOCK_B, PROMPT_LEN, HIDDEN),
                               lambda b, ids: (b, 0, 0)),
    )
    return pl.pallas_call(
        body,
        grid_spec=grid_spec,
        out_shape=jax.ShapeDtypeStruct((BATCH, PROMPT_LEN, HIDDEN),
                                       jnp.float32),
    )(task_ids, table)


def _sc_part(ids_pad, table):
    mesh = plsc.VectorSubcoreMesh(core_axis_name="c", subcore_axis_name="s")

    @functools.partial(
        pl.kernel,
        out_type=jax.ShapeDtypeStruct((B_SC, PROMPT_LEN, HIDDEN), jnp.float32),
        mesh=mesh,
        scratch_types=[
            pltpu.VMEM((16,), jnp.int32),
            pltpu.VMEM_SHARED((NUM_TASKS, PROMPT_LEN, HIDDEN), jnp.float32),
            pltpu.SemaphoreType.DMA,
        ],
    )
    def run(idx_hbm, table_hbm, out_hbm, idx_v, sh_table, sem):
        c = lax.axis_index("c")
        s = lax.axis_index("s")
        wid = s * NUM_CORES + c
        base = wid * B_PER_TILE
        pltpu.sync_copy(idx_hbm.at[pl.ds(base, 16)], idx_v)

        @pl.when(s == 0)
        def _():
            pltpu.sync_copy(table_hbm, sh_table)

        plsc.subcore_barrier()
        vec = idx_v[pl.ds(0, 16)]
        for i in range(B_PER_TILE):
            tid = vec[i]
            pltpu.async_copy(sh_table.at[tid], out_hbm.at[base + i], sem)
        for _ in range(B_PER_TILE):
            pltpu.make_async_copy(
                sh_table.at[0], out_hbm.at[base], sem).wait()

    return run(ids_pad, table)


def _tc_merge(big, sc_part):
    def body(big_ref, sc_ref, out_ref):
        out_ref[...] = sc_ref[...]

    return pl.pallas_call(
        body,
        grid=(B_SC // BLOCK_B,),
        in_specs=[
            pl.BlockSpec(memory_space=pl.ANY),
            pl.BlockSpec((BLOCK_B, PROMPT_LEN, HIDDEN),
                         lambda b: (b, 0, 0)),
        ],
        out_specs=pl.BlockSpec((BLOCK_B, PROMPT_LEN, HIDDEN),
                               lambda b: (B_TC // BLOCK_B + b, 0, 0)),
        out_shape=jax.ShapeDtypeStruct((BATCH, PROMPT_LEN, HIDDEN),
                                       jnp.float32),
        input_output_aliases={0: 0},
    )(big, sc_part)


def kernel(task_ids, prompt_embeddings):
    ids = task_ids.astype(jnp.int32)
    ids_pad = jnp.concatenate(
        [ids[B_TC:], jnp.zeros((16,), jnp.int32)])
    big = _tc_main(ids, prompt_embeddings)
    sc_out = _sc_part(ids_pad, prompt_embeddings)
    return _tc_merge(big, sc_out)


# speedup vs baseline: 1.7605x; 1.1027x over previous
import functools

import jax
import jax.numpy as jnp
from jax import lax
from jax.experimental import pallas as pl
from jax.experimental.pallas import tpu as pltpu
from jax.experimental.pallas import tpu_sc as plsc

NUM_TASKS = 3
PROMPT_LEN = 20
HIDDEN = 4096
BATCH = 1024

B_TC = 768
B_SC = BATCH - B_TC

NUM_CORES = 2
NUM_SUBCORES = 16
NUM_WORKERS = NUM_CORES * NUM_SUBCORES

B_PER_TILE = B_SC // NUM_WORKERS
BLOCK_B = 8


def _tc_main(task_ids, table):
    def body(ids_ref, table_ref, out_ref):
        b0 = pl.program_id(0) * BLOCK_B
        for i in range(BLOCK_B):
            tid = ids_ref[b0 + i]
            out_ref[i] = table_ref[tid]

    grid_spec = pltpu.PrefetchScalarGridSpec(
        num_scalar_prefetch=1,
        grid=(B_TC // BLOCK_B,),
        in_specs=[
            pl.BlockSpec((NUM_TASKS, PROMPT_LEN, HIDDEN),
                         lambda b, ids: (0, 0, 0)),
        ],
        out_specs=pl.BlockSpec((BLOCK_B, PROMPT_LEN, HIDDEN),
                               lambda b, ids: (b, 0, 0)),
    )
    return pl.pallas_call(
        body,
        grid_spec=grid_spec,
        out_shape=jax.ShapeDtypeStruct((BATCH, PROMPT_LEN, HIDDEN),
                                       jnp.float32),
    )(task_ids, table)


def _sc_part(ids_pad, table):
    mesh = plsc.VectorSubcoreMesh(core_axis_name="c", subcore_axis_name="s")

    @functools.partial(
        pl.kernel,
        out_type=jax.ShapeDtypeStruct((B_SC, PROMPT_LEN, HIDDEN), jnp.float32),
        mesh=mesh,
        scratch_types=[
            pltpu.VMEM((16,), jnp.int32),
            pltpu.VMEM_SHARED((NUM_TASKS, PROMPT_LEN, HIDDEN), jnp.float32),
            pltpu.SemaphoreType.DMA,
        ],
    )
    def run(idx_hbm, table_hbm, out_hbm, idx_v, sh_table, sem):
        c = lax.axis_index("c")
        s = lax.axis_index("s")
        wid = s * NUM_CORES + c
        base = wid * B_PER_TILE
        pltpu.sync_copy(idx_hbm.at[pl.ds(base, 16)], idx_v)

        @pl.when(s == 0)
        def _():
            pltpu.sync_copy(table_hbm, sh_table)

        plsc.subcore_barrier()
        vec = idx_v[pl.ds(0, 16)]
        for i in range(B_PER_TILE):
            tid = vec[i]
            pltpu.async_copy(sh_table.at[tid], out_hbm.at[base + i], sem)
        for _ in range(B_PER_TILE):
            pltpu.make_async_copy(
                sh_table.at[0], out_hbm.at[base], sem).wait()

    return run(ids_pad, table)


def _tc_merge(big, sc_part):
    def body(big_ref, sc_ref, out_ref):
        out_ref[...] = sc_ref[...]

    return pl.pallas_call(
        body,
        grid=(B_SC // BLOCK_B,),
        in_specs=[
            pl.BlockSpec(memory_space=pl.ANY),
            pl.BlockSpec((BLOCK_B, PROMPT_LEN, HIDDEN),
                         lambda b: (b, 0, 0)),
        ],
        out_specs=pl.BlockSpec((BLOCK_B, PROMPT_LEN, HIDDEN),
                               lambda b: (B_TC // BLOCK_B + b, 0, 0)),
        out_shape=jax.ShapeDtypeStruct((BATCH, PROMPT_LEN, HIDDEN),
                                       jnp.float32),
        input_output_aliases={0: 0},
    )(big, sc_part)


def kernel(task_ids, prompt_embeddings):
    ids = task_ids.astype(jnp.int32)
    ids_pad = jnp.concatenate(
        [ids[B_TC:], jnp.zeros((16,), jnp.int32)])
    big = _tc_main(ids, prompt_embeddings)
    sc_out = _sc_part(ids_pad, prompt_embeddings)
    return lax.dynamic_update_slice(big, sc_out, (B_TC, 0, 0))
